# baseline (speedup 1.0000x reference)
import functools

import numpy as np
import jax
import jax.numpy as jnp
from jax import lax
from jax.experimental import pallas as pl
from jax.experimental.pallas import tpu as pltpu
from jax.experimental.pallas import tpu_sc as plsc

NODE_EMB = 64
EDGE_EMB = 64
D_MODEL = 128
WPR = NODE_EMB // 2
LANES = 128
ROWS_PER_GRP = 4
GRP = LANES * ROWS_PER_GRP
N_CHUNKS = 2
BB = 32
_TCB = 512


def _pe_table(seq_len, d_model):
    pos = np.arange(seq_len, dtype=np.float32)[:, None]
    div = np.exp(np.arange(0, d_model, 2, dtype=np.float32)
                 * (-np.log(10000.0) / d_model))
    pe = np.zeros((seq_len, d_model), dtype=np.float32)
    pe[:, 0::2] = np.sin(pos * div)
    pe[:, 1::2] = np.cos(pos * div)
    return pe


def _rne_bf16(x):
    u = lax.bitcast_convert_type(x, jnp.int32)
    return lax.shift_right_logical(
        u + 0x7FFF + (lax.shift_right_logical(u, 16) & 1), 16)


def _transp_body(a0, a1, a2, a3, o_ref):
    for k, a in enumerate((a0, a1, a2, a3)):
        t = a[...].T
        word = _rne_bf16(t[:, :WPR]) | (_rne_bf16(t[:, WPR:]) << 16)
        o_ref[:, k * WPR:(k + 1) * WPR] = word


def _relayout_table(table):
    V = table.shape[0]
    grid_n = (V + 4 * _TCB - 1) // (4 * _TCB)
    max_blk = (V + _TCB - 1) // _TCB - 1
    tT = table.T
    specs = [
        pl.BlockSpec((NODE_EMB, _TCB),
                     lambda i, k=k, m=max_blk: (0, jnp.minimum(4 * i + k, m)))
        for k in range(4)
    ]
    packed = pl.pallas_call(
        _transp_body,
        grid=(grid_n,),
        in_specs=specs,
        out_specs=pl.BlockSpec((_TCB, 4 * WPR), lambda i: (i, 0)),
        out_shape=jax.ShapeDtypeStruct((grid_n * _TCB, 4 * WPR), jnp.int32),
    )(tT, tT, tT, tT)
    return packed.reshape(grid_n * 4 * _TCB, WPR)


def _remap_idx(idx):
    r = idx & (4 * _TCB - 1)
    return (idx - r) + ((r & (_TCB - 1)) << 2) + (r >> 9)


def _sc_gather(n_idx_rows, n_tokens):
    info = plsc.get_sparse_core_info()
    nc, ns = info.num_cores, info.num_subcores
    nw = nc * ns
    rows_per_w = n_idx_rows // nw
    grps = rows_per_w // ROWS_PER_GRP
    mesh = plsc.VectorSubcoreMesh(core_axis_name="c", subcore_axis_name="s")

    @functools.partial(
        pl.kernel, mesh=mesh,
        out_type=jax.ShapeDtypeStruct((n_tokens, 2 * WPR), jnp.int32),
        scratch_types=[
            pltpu.VMEM((ROWS_PER_GRP, LANES), jnp.int32),
            pltpu.VMEM((ROWS_PER_GRP, LANES), jnp.int32),
            pltpu.VMEM((GRP, WPR), jnp.int32),
            pltpu.VMEM((GRP, WPR), jnp.int32),
            pltpu.SemaphoreType.DMA,
        ],
        compiler_params=pltpu.CompilerParams(use_tc_tiling_on_sc=False),
    )
    def gather_k(vidx_hbm, eidx_hbm, ntab_hbm, etab_hbm,
                 cat_hbm, vidx, eidx, vrows, erows, sem):
        wid = lax.axis_index("s") * nc + lax.axis_index("c")
        row0 = wid * rows_per_w

        def body(g, carry):
            r = row0 + g * ROWS_PER_GRP
            pltpu.sync_copy(vidx_hbm.at[pl.ds(r, ROWS_PER_GRP)], vidx)
            pltpu.sync_copy(eidx_hbm.at[pl.ds(r, ROWS_PER_GRP)], eidx)
            cps = []
            for j in range(ROWS_PER_GRP):
                cps.append(pltpu.async_copy(
                    ntab_hbm.at[vidx.at[j]],
                    vrows.at[pl.ds(j * LANES, LANES)], sem))
                cps.append(pltpu.async_copy(
                    etab_hbm.at[eidx.at[j]],
                    erows.at[pl.ds(j * LANES, LANES)], sem))
            for c in cps:
                c.wait()
            tok = r * LANES
            pltpu.sync_copy(
                vrows, cat_hbm.at[pl.ds(tok, GRP), pl.ds(0, WPR)])
            pltpu.sync_copy(
                erows, cat_hbm.at[pl.ds(tok, GRP), pl.ds(WPR, WPR)])
            return carry

        lax.fori_loop(0, grps, body, 0)

    return gather_k


def _unpack_dot(x, wlo_ref, whi_ref, b_ref):
    lo = lax.bitcast_convert_type(x << 16, jnp.float32)
    hi = lax.bitcast_convert_type(x & jnp.int32(-65536), jnp.float32)
    h = jnp.dot(lo, wlo_ref[...], preferred_element_type=jnp.float32)
    h = h + jnp.dot(hi, whi_ref[...], preferred_element_type=jnp.float32)
    return h + b_ref[...]


def _tc_compute(cat_ref, wlo_ref, whi_ref, b_ref, pea_ref, peb_ref,
                h_ref, hp_ref):
    x = cat_ref[...]
    ha = _unpack_dot(x[:, 0:2 * WPR], wlo_ref, whi_ref, b_ref)
    hb = _unpack_dot(x[:, 2 * WPR:], wlo_ref, whi_ref, b_ref)
    h_ref[:, 0:D_MODEL] = ha
    h_ref[:, D_MODEL:] = hb
    hp_ref[:, 0:D_MODEL] = ha + pea_ref[...]
    hp_ref[:, D_MODEL:] = hb + peb_ref[...]


def _tc_body_first(cat_ref, wlo_ref, whi_ref, b_ref, pea_ref, peb_ref,
                   h_ref, hp_ref):
    _tc_compute(cat_ref, wlo_ref, whi_ref, b_ref, pea_ref, peb_ref,
                h_ref, hp_ref)


def _tc_body_next(cat_ref, wlo_ref, whi_ref, b_ref, pea_ref, peb_ref,
                  hin_ref, hpin_ref, h_ref, hp_ref):
    del hin_ref, hpin_ref
    _tc_compute(cat_ref, wlo_ref, whi_ref, b_ref, pea_ref, peb_ref,
                h_ref, hp_ref)


def kernel(v_list, e_list, node_table, edge_table, W, b):
    B, L = v_list.shape
    n_tokens = B * L
    n_idx_rows = n_tokens // LANES
    hL = L // 2

    nt_lin = _relayout_table(node_table)
    et_lin = _relayout_table(edge_table)

    v2d = _remap_idx(v_list.reshape(n_idx_rows, LANES))
    e2d = _remap_idx(e_list.reshape(n_idx_rows, LANES))

    rows_c = n_idx_rows // N_CHUNKS
    tok_c = n_tokens // N_CHUNKS
    batch_c = B // N_CHUNKS
    nblk_c = batch_c // BB
    RB = BB * hL

    gk = _sc_gather(rows_c, tok_c)
    cats = [
        gk(v2d[k * rows_c:(k + 1) * rows_c],
           e2d[k * rows_c:(k + 1) * rows_c],
           nt_lin, et_lin).reshape(tok_c // 2, D_MODEL)
        for k in range(N_CHUNKS)
    ]

    w2 = (np.sqrt(float(NODE_EMB)) * W).T
    lo_rows = np.r_[0:WPR, NODE_EMB:NODE_EMB + WPR]
    hi_rows = np.r_[WPR:NODE_EMB, NODE_EMB + WPR:2 * NODE_EMB]
    wlo = w2[lo_rows, :]
    whi = w2[hi_rows, :]
    b2 = b.reshape(1, D_MODEL)
    pe = _pe_table(L, D_MODEL)
    pea = jnp.asarray(np.tile(pe[0::2], (BB, 1)))
    peb = jnp.asarray(np.tile(pe[1::2], (BB, 1)))

    n_pairs = n_tokens // 2
    out_shape = [
        jax.ShapeDtypeStruct((n_pairs, 2 * D_MODEL), jnp.float32),
        jax.ShapeDtypeStruct((n_pairs, 2 * D_MODEL), jnp.float32),
    ]
    common_specs = [
        pl.BlockSpec((NODE_EMB, D_MODEL), lambda i: (0, 0)),
        pl.BlockSpec((NODE_EMB, D_MODEL), lambda i: (0, 0)),
        pl.BlockSpec((1, D_MODEL), lambda i: (0, 0)),
        pl.BlockSpec((RB, D_MODEL), lambda i: (0, 0)),
        pl.BlockSpec((RB, D_MODEL), lambda i: (0, 0)),
    ]

    h_emb, h_pos = pl.pallas_call(
        _tc_body_first,
        grid=(nblk_c,),
        in_specs=[pl.BlockSpec((RB, D_MODEL), lambda i: (i, 0))]
        + common_specs,
        out_specs=[
            pl.BlockSpec((RB, 2 * D_MODEL), lambda i: (i, 0)),
            pl.BlockSpec((RB, 2 * D_MODEL), lambda i: (i, 0)),
        ],
        out_shape=out_shape,
    )(cats[0], wlo, whi, b2, pea, peb)

    for k in range(1, N_CHUNKS):
        off = k * nblk_c
        h_emb, h_pos = pl.pallas_call(
            _tc_body_next,
            grid=(nblk_c,),
            in_specs=[pl.BlockSpec((RB, D_MODEL), lambda i: (i, 0))]
            + common_specs
            + [pl.BlockSpec(memory_space=pl.ANY),
               pl.BlockSpec(memory_space=pl.ANY)],
            out_specs=[
                pl.BlockSpec((RB, 2 * D_MODEL),
                             lambda i, off=off: (i + off, 0)),
                pl.BlockSpec((RB, 2 * D_MODEL),
                             lambda i, off=off: (i + off, 0)),
            ],
            out_shape=out_shape,
            input_output_aliases={6: 0, 7: 1},
        )(cats[k], wlo, whi, b2, pea, peb, h_emb, h_pos)

    return (h_emb.reshape(B, L, D_MODEL), h_pos.reshape(B, L, D_MODEL))


# speedup vs baseline: 1.8530x; 1.8530x over previous
import functools

import numpy as np
import jax
import jax.numpy as jnp
from jax import lax
from jax.experimental import pallas as pl
from jax.experimental.pallas import tpu as pltpu
from jax.experimental.pallas import tpu_sc as plsc

NODE_EMB = 64
EDGE_EMB = 64
D_MODEL = 128
WPR = NODE_EMB // 2
LANES = 128
ROWS_PER_GRP = 4
GRP = LANES * ROWS_PER_GRP
N_CHUNKS = 2
BB = 16
RB = BB * 200
_TCB = 1024


def _pe_table(seq_len, d_model):
    pos = np.arange(seq_len, dtype=np.float32)[:, None]
    div = np.exp(np.arange(0, d_model, 2, dtype=np.float32)
                 * (-np.log(10000.0) / d_model))
    pe = np.zeros((seq_len, d_model), dtype=np.float32)
    pe[:, 0::2] = np.sin(pos * div)
    pe[:, 1::2] = np.cos(pos * div)
    return pe


def _rne_bf16(x):
    u = lax.bitcast_convert_type(x, jnp.int32)
    return lax.shift_right_logical(
        u + 0x7FFF + (lax.shift_right_logical(u, 16) & 1), 16)


def _transp_body(a0, a1, a2, a3, o_ref):
    for k, a in enumerate((a0, a1, a2, a3)):
        blk = a[...]
        word = _rne_bf16(blk[:WPR, :]) | (_rne_bf16(blk[WPR:, :]) << 16)
        o_ref[:, k * WPR:(k + 1) * WPR] = word.T


def _relayout_table(table):
    V = table.shape[0]
    grid_n = (V + 4 * _TCB - 1) // (4 * _TCB)
    max_blk = (V + _TCB - 1) // _TCB - 1
    tT = table.T
    specs = [
        pl.BlockSpec((NODE_EMB, _TCB),
                     lambda i, k=k, m=max_blk: (0, jnp.minimum(4 * i + k, m)))
        for k in range(4)
    ]
    packed = pl.pallas_call(
        _transp_body,
        grid=(grid_n,),
        in_specs=specs,
        out_specs=pl.BlockSpec((_TCB, 4 * WPR), lambda i: (i, 0)),
        out_shape=jax.ShapeDtypeStruct((grid_n * _TCB, 4 * WPR), jnp.int32),
    )(tT, tT, tT, tT)
    return packed.reshape(grid_n * 4 * _TCB, WPR)


def _remap_idx(idx):
    r = idx & (4 * _TCB - 1)
    return (idx - r) + ((r & (_TCB - 1)) << 2) + (r >> 10)


def _sc_gather(n_idx_rows, n_tokens):
    info = plsc.get_sparse_core_info()
    nc, ns = info.num_cores, info.num_subcores
    nw = nc * ns
    rows_per_w = n_idx_rows // nw
    grps = rows_per_w // ROWS_PER_GRP
    half_w = nw // 2
    half_rows = n_idx_rows // 2
    mesh = plsc.VectorSubcoreMesh(core_axis_name="c", subcore_axis_name="s")

    @functools.partial(
        pl.kernel, mesh=mesh,
        out_type=jax.ShapeDtypeStruct((n_tokens // 2, 4 * WPR), jnp.int32),
        scratch_types=[
            pltpu.VMEM((ROWS_PER_GRP, LANES), jnp.int32),
            pltpu.VMEM((ROWS_PER_GRP, LANES), jnp.int32),
            pltpu.VMEM((GRP, WPR), jnp.int32),
            pltpu.VMEM((GRP, WPR), jnp.int32),
            pltpu.SemaphoreType.DMA,
        ],
        compiler_params=pltpu.CompilerParams(use_tc_tiling_on_sc=False),
    )
    def gather_k(vidx_hbm, eidx_hbm, ntab_hbm, etab_hbm,
                 cat_hbm, vidx, eidx, vrows, erows, sem):
        wid = lax.axis_index("s") * nc + lax.axis_index("c")
        row0 = wid * rows_per_w
        in_b = (wid >= half_w).astype(jnp.int32)
        col0 = in_b * (2 * WPR)
        out_row0 = (row0 - in_b * half_rows) * LANES

        def body(g, carry):
            r = row0 + g * ROWS_PER_GRP
            pltpu.sync_copy(vidx_hbm.at[pl.ds(r, ROWS_PER_GRP)], vidx)
            pltpu.sync_copy(eidx_hbm.at[pl.ds(r, ROWS_PER_GRP)], eidx)
            cps = []
            for j in range(ROWS_PER_GRP):
                cps.append(pltpu.async_copy(
                    ntab_hbm.at[vidx.at[j]],
                    vrows.at[pl.ds(j * LANES, LANES)], sem))
                cps.append(pltpu.async_copy(
                    etab_hbm.at[eidx.at[j]],
                    erows.at[pl.ds(j * LANES, LANES)], sem))
            for c in cps:
                c.wait()
            tok = out_row0 + g * GRP
            pltpu.sync_copy(
                vrows, cat_hbm.at[pl.ds(tok, GRP), pl.ds(col0, WPR)])
            pltpu.sync_copy(
                erows, cat_hbm.at[pl.ds(tok, GRP), pl.ds(col0 + WPR, WPR)])
            return carry

        lax.fori_loop(0, grps, body, 0)

    return gather_k


def _unpack_dot(x, wlo_ref, whi_ref, b_ref):
    lo = lax.bitcast_convert_type(x << 16, jnp.float32)
    hi = lax.bitcast_convert_type(x & jnp.int32(-65536), jnp.float32)
    h = jnp.dot(lo, wlo_ref[...], preferred_element_type=jnp.float32)
    h = h + jnp.dot(hi, whi_ref[...], preferred_element_type=jnp.float32)
    return h + b_ref[...]


def _tc_compute(x, wlo_ref, whi_ref, b_ref, pe_ref, h_ref, hp_ref):
    h = _unpack_dot(x, wlo_ref, whi_ref, b_ref)
    h_ref[...] = h
    hp_ref[...] = h + pe_ref[...]


def _tc_body_a0(cat_ref, wlo_ref, whi_ref, b_ref, pe_ref, h_ref, hp_ref):
    _tc_compute(cat_ref[:, 0:2 * WPR], wlo_ref, whi_ref, b_ref, pe_ref,
                h_ref, hp_ref)


def _tc_body_a(cat_ref, wlo_ref, whi_ref, b_ref, pe_ref, hin, hpin,
               h_ref, hp_ref):
    del hin, hpin
    _tc_compute(cat_ref[:, 0:2 * WPR], wlo_ref, whi_ref, b_ref, pe_ref,
                h_ref, hp_ref)


def _tc_body_b(cat_ref, wlo_ref, whi_ref, b_ref, pe_ref, hin, hpin,
               h_ref, hp_ref):
    del hin, hpin
    _tc_compute(cat_ref[:, 2 * WPR:], wlo_ref, whi_ref, b_ref, pe_ref,
                h_ref, hp_ref)


def kernel(v_list, e_list, node_table, edge_table, W, b):
    B, L = v_list.shape
    n_tokens = B * L
    n_idx_rows = n_tokens // LANES

    nt_lin = _relayout_table(node_table)
    et_lin = _relayout_table(edge_table)

    v2d = _remap_idx(v_list.reshape(n_idx_rows, LANES))
    e2d = _remap_idx(e_list.reshape(n_idx_rows, LANES))

    rows_c = n_idx_rows // N_CHUNKS
    tok_c = n_tokens // N_CHUNKS
    nblk_half = tok_c // 2 // RB

    gk = _sc_gather(rows_c, tok_c)
    cats = [
        gk(v2d[k * rows_c:(k + 1) * rows_c],
           e2d[k * rows_c:(k + 1) * rows_c],
           nt_lin, et_lin)
        for k in range(N_CHUNKS)
    ]

    w2 = (np.sqrt(float(NODE_EMB)) * W).T
    lo_rows = np.r_[0:WPR, NODE_EMB:NODE_EMB + WPR]
    hi_rows = np.r_[WPR:NODE_EMB, NODE_EMB + WPR:2 * NODE_EMB]
    wlo = w2[lo_rows, :]
    whi = w2[hi_rows, :]
    b2 = b.reshape(1, D_MODEL)
    pe_tile = jnp.asarray(np.tile(_pe_table(L, D_MODEL), (BB, 1)))

    out_shape = [
        jax.ShapeDtypeStruct((n_tokens, D_MODEL), jnp.float32),
        jax.ShapeDtypeStruct((n_tokens, D_MODEL), jnp.float32),
    ]
    common_specs = [
        pl.BlockSpec((NODE_EMB, D_MODEL), lambda i: (0, 0)),
        pl.BlockSpec((NODE_EMB, D_MODEL), lambda i: (0, 0)),
        pl.BlockSpec((1, D_MODEL), lambda i: (0, 0)),
        pl.BlockSpec((RB, D_MODEL), lambda i: (0, 0)),
    ]
    alias_specs = [pl.BlockSpec(memory_space=pl.ANY),
                   pl.BlockSpec(memory_space=pl.ANY)]

    h_emb = h_pos = None
    for k in range(N_CHUNKS):
        for phase, body in ((0, _tc_body_a if k else _tc_body_a0),
                            (1, _tc_body_b)):
            off = (2 * k + phase) * nblk_half
            first = h_emb is None
            outs = pl.pallas_call(
                body,
                grid=(nblk_half,),
                in_specs=[pl.BlockSpec((RB, D_MODEL), lambda i: (i, 0))]
                + common_specs + ([] if first else alias_specs),
                out_specs=[
                    pl.BlockSpec((RB, D_MODEL),
                                 lambda i, off=off: (i + off, 0)),
                    pl.BlockSpec((RB, D_MODEL),
                                 lambda i, off=off: (i + off, 0)),
                ],
                out_shape=out_shape,
                input_output_aliases={} if first else {5: 0, 6: 1},
            )(*([cats[k], wlo, whi, b2, pe_tile]
                + ([] if first else [h_emb, h_pos])))
            h_emb, h_pos = outs

    return (h_emb.reshape(B, L, D_MODEL), h_pos.reshape(B, L, D_MODEL))


# speedup vs baseline: 2.0275x; 1.0942x over previous
import functools

import numpy as np
import jax
import jax.numpy as jnp
from jax import lax
from jax.experimental import pallas as pl
from jax.experimental.pallas import tpu as pltpu
from jax.experimental.pallas import tpu_sc as plsc

NODE_EMB = 64
EDGE_EMB = 64
D_MODEL = 128
WPR = NODE_EMB // 2
LANES = 128
ROWS_PER_GRP = 5
GRP = LANES * ROWS_PER_GRP
N_CHUNKS = 4
BB = 16
RB = BB * 200
_TCB = 2048
_TCB_SHIFT = _TCB.bit_length() - 1


def _pe_table(seq_len, d_model):
    pos = np.arange(seq_len, dtype=np.float32)[:, None]
    div = np.exp(np.arange(0, d_model, 2, dtype=np.float32)
                 * (-np.log(10000.0) / d_model))
    pe = np.zeros((seq_len, d_model), dtype=np.float32)
    pe[:, 0::2] = np.sin(pos * div)
    pe[:, 1::2] = np.cos(pos * div)
    return pe


def _rne_bf16(x):
    u = lax.bitcast_convert_type(x, jnp.int32)
    return lax.shift_right_logical(
        u + 0x7FFF + (lax.shift_right_logical(u, 16) & 1), 16)


def _transp_body(a0, a1, a2, a3, o_ref):
    for k, a in enumerate((a0, a1, a2, a3)):
        blk = a[...]
        word = _rne_bf16(blk[:WPR, :]) | (_rne_bf16(blk[WPR:, :]) << 16)
        o_ref[:, k * WPR:(k + 1) * WPR] = word.T


def _relayout_table(table):
    V = table.shape[0]
    grid_n = (V + 4 * _TCB - 1) // (4 * _TCB)
    max_blk = (V + _TCB - 1) // _TCB - 1
    tT = table.T
    specs = [
        pl.BlockSpec((NODE_EMB, _TCB),
                     lambda i, k=k, m=max_blk: (0, jnp.minimum(4 * i + k, m)))
        for k in range(4)
    ]
    packed = pl.pallas_call(
        _transp_body,
        grid=(grid_n,),
        in_specs=specs,
        out_specs=pl.BlockSpec((_TCB, 4 * WPR), lambda i: (i, 0)),
        out_shape=jax.ShapeDtypeStruct((grid_n * _TCB, 4 * WPR), jnp.int32),
    )(tT, tT, tT, tT)
    return packed.reshape(grid_n * 4 * _TCB, WPR)


def _remap_idx(idx):
    r = idx & (4 * _TCB - 1)
    return (idx - r) + ((r & (_TCB - 1)) << 2) + (r >> _TCB_SHIFT)


def _sc_gather(n_idx_rows, n_tokens):
    info = plsc.get_sparse_core_info()
    nc, ns = info.num_cores, info.num_subcores
    nw = nc * ns
    rows_per_w = n_idx_rows // nw
    grps = rows_per_w // ROWS_PER_GRP
    half_w = nw // 2
    half_rows = n_idx_rows // 2
    mesh = plsc.VectorSubcoreMesh(core_axis_name="c", subcore_axis_name="s")

    @functools.partial(
        pl.kernel, mesh=mesh,
        out_type=jax.ShapeDtypeStruct((n_tokens // 2, 4 * WPR), jnp.int32),
        scratch_types=[
            pltpu.VMEM((ROWS_PER_GRP, LANES), jnp.int32),
            pltpu.VMEM((ROWS_PER_GRP, LANES), jnp.int32),
            pltpu.VMEM((GRP, WPR), jnp.int32),
            pltpu.VMEM((GRP, WPR), jnp.int32),
            pltpu.SemaphoreType.DMA,
        ],
        compiler_params=pltpu.CompilerParams(use_tc_tiling_on_sc=False),
    )
    def gather_k(vidx_hbm, eidx_hbm, ntab_hbm, etab_hbm,
                 cat_hbm, vidx, eidx, vrows, erows, sem):
        wid = lax.axis_index("s") * nc + lax.axis_index("c")
        row0 = wid * rows_per_w
        in_b = (wid >= half_w).astype(jnp.int32)
        col0 = in_b * (2 * WPR)
        out_row0 = (row0 - in_b * half_rows) * LANES

        def body(g, carry):
            r = row0 + g * ROWS_PER_GRP
            pltpu.sync_copy(vidx_hbm.at[pl.ds(r, ROWS_PER_GRP)], vidx)
            pltpu.sync_copy(eidx_hbm.at[pl.ds(r, ROWS_PER_GRP)], eidx)
            cps = []
            for j in range(ROWS_PER_GRP):
                cps.append(pltpu.async_copy(
                    ntab_hbm.at[vidx.at[j]],
                    vrows.at[pl.ds(j * LANES, LANES)], sem))
                cps.append(pltpu.async_copy(
                    etab_hbm.at[eidx.at[j]],
                    erows.at[pl.ds(j * LANES, LANES)], sem))
            for c in cps:
                c.wait()
            tok = out_row0 + g * GRP
            pltpu.sync_copy(
                vrows, cat_hbm.at[pl.ds(tok, GRP), pl.ds(col0, WPR)])
            pltpu.sync_copy(
                erows, cat_hbm.at[pl.ds(tok, GRP), pl.ds(col0 + WPR, WPR)])
            return carry

        lax.fori_loop(0, grps, body, 0)

    return gather_k


def _unpack_dot(x, wlo_ref, whi_ref, b_ref):
    lo = lax.bitcast_convert_type(x << 16, jnp.float32)
    hi = lax.bitcast_convert_type(x & jnp.int32(-65536), jnp.float32)
    h = jnp.dot(lo, wlo_ref[...], preferred_element_type=jnp.float32)
    h = h + jnp.dot(hi, whi_ref[...], preferred_element_type=jnp.float32)
    return h + b_ref[...]


def _tc_compute(x, wlo_ref, whi_ref, b_ref, pe_ref, h_ref, hp_ref):
    h = _unpack_dot(x, wlo_ref, whi_ref, b_ref)
    h_ref[...] = h
    hp_ref[...] = h + pe_ref[...]


def _tc_body_a0(cat_ref, wlo_ref, whi_ref, b_ref, pe_ref, h_ref, hp_ref):
    _tc_compute(cat_ref[:, 0:2 * WPR], wlo_ref, whi_ref, b_ref, pe_ref,
                h_ref, hp_ref)


def _tc_body_a(cat_ref, wlo_ref, whi_ref, b_ref, pe_ref, hin, hpin,
               h_ref, hp_ref):
    del hin, hpin
    _tc_compute(cat_ref[:, 0:2 * WPR], wlo_ref, whi_ref, b_ref, pe_ref,
                h_ref, hp_ref)


def _tc_body_b(cat_ref, wlo_ref, whi_ref, b_ref, pe_ref, hin, hpin,
               h_ref, hp_ref):
    del hin, hpin
    _tc_compute(cat_ref[:, 2 * WPR:], wlo_ref, whi_ref, b_ref, pe_ref,
                h_ref, hp_ref)


def kernel(v_list, e_list, node_table, edge_table, W, b):
    B, L = v_list.shape
    n_tokens = B * L
    n_idx_rows = n_tokens // LANES

    nt_lin = _relayout_table(node_table)
    et_lin = _relayout_table(edge_table)

    v2d = _remap_idx(v_list.reshape(n_idx_rows, LANES))
    e2d = _remap_idx(e_list.reshape(n_idx_rows, LANES))

    rows_c = n_idx_rows // N_CHUNKS
    tok_c = n_tokens // N_CHUNKS
    nblk_half = tok_c // 2 // RB

    gk = _sc_gather(rows_c, tok_c)
    cats = [
        gk(v2d[k * rows_c:(k + 1) * rows_c],
           e2d[k * rows_c:(k + 1) * rows_c],
           nt_lin, et_lin)
        for k in range(N_CHUNKS)
    ]

    w2 = (np.sqrt(float(NODE_EMB)) * W).T
    lo_rows = np.r_[0:WPR, NODE_EMB:NODE_EMB + WPR]
    hi_rows = np.r_[WPR:NODE_EMB, NODE_EMB + WPR:2 * NODE_EMB]
    wlo = w2[lo_rows, :]
    whi = w2[hi_rows, :]
    b2 = b.reshape(1, D_MODEL)
    pe_tile = jnp.asarray(np.tile(_pe_table(L, D_MODEL), (BB, 1)))

    out_shape = [
        jax.ShapeDtypeStruct((n_tokens, D_MODEL), jnp.float32),
        jax.ShapeDtypeStruct((n_tokens, D_MODEL), jnp.float32),
    ]
    common_specs = [
        pl.BlockSpec((NODE_EMB, D_MODEL), lambda i: (0, 0)),
        pl.BlockSpec((NODE_EMB, D_MODEL), lambda i: (0, 0)),
        pl.BlockSpec((1, D_MODEL), lambda i: (0, 0)),
        pl.BlockSpec((RB, D_MODEL), lambda i: (0, 0)),
    ]
    alias_specs = [pl.BlockSpec(memory_space=pl.ANY),
                   pl.BlockSpec(memory_space=pl.ANY)]

    h_emb = h_pos = None
    for k in range(N_CHUNKS):
        for phase, body in ((0, _tc_body_a if k else _tc_body_a0),
                            (1, _tc_body_b)):
            off = (2 * k + phase) * nblk_half
            first = h_emb is None
            outs = pl.pallas_call(
                body,
                grid=(nblk_half,),
                in_specs=[pl.BlockSpec((RB, D_MODEL), lambda i: (i, 0))]
                + common_specs + ([] if first else alias_specs),
                out_specs=[
                    pl.BlockSpec((RB, D_MODEL),
                                 lambda i, off=off: (i + off, 0)),
                    pl.BlockSpec((RB, D_MODEL),
                                 lambda i, off=off: (i + off, 0)),
                ],
                out_shape=out_shape,
                input_output_aliases={} if first else {5: 0, 6: 1},
            )(*([cats[k], wlo, whi, b2, pe_tile]
                + ([] if first else [h_emb, h_pos])))
            h_emb, h_pos = outs

    return (h_emb.reshape(B, L, D_MODEL), h_pos.reshape(B, L, D_MODEL))


# speedup vs baseline: 2.0359x; 1.0042x over previous
import functools

import numpy as np
import jax
import jax.numpy as jnp
from jax import lax
from jax.experimental import pallas as pl
from jax.experimental.pallas import tpu as pltpu
from jax.experimental.pallas import tpu_sc as plsc

NODE_EMB = 64
EDGE_EMB = 64
D_MODEL = 128
WPR = NODE_EMB // 2
LANES = 128
ROWS_PER_GRP = 5
GRP = LANES * ROWS_PER_GRP
N_CHUNKS = 8
BB = 16
RB = BB * 200
_TCB = 4096
_TCB_SHIFT = _TCB.bit_length() - 1


def _pe_table(seq_len, d_model):
    pos = np.arange(seq_len, dtype=np.float32)[:, None]
    div = np.exp(np.arange(0, d_model, 2, dtype=np.float32)
                 * (-np.log(10000.0) / d_model))
    pe = np.zeros((seq_len, d_model), dtype=np.float32)
    pe[:, 0::2] = np.sin(pos * div)
    pe[:, 1::2] = np.cos(pos * div)
    return pe


def _rne_bf16(x):
    u = lax.bitcast_convert_type(x, jnp.int32)
    return lax.shift_right_logical(
        u + 0x7FFF + (lax.shift_right_logical(u, 16) & 1), 16)


def _transp_body(a0, a1, a2, a3, o_ref):
    for k, a in enumerate((a0, a1, a2, a3)):
        blk = a[...]
        word = _rne_bf16(blk[:WPR, :]) | (_rne_bf16(blk[WPR:, :]) << 16)
        o_ref[:, k * WPR:(k + 1) * WPR] = word.T


def _relayout_table(table):
    V = table.shape[0]
    grid_n = (V + 4 * _TCB - 1) // (4 * _TCB)
    max_blk = (V + _TCB - 1) // _TCB - 1
    tT = table.T
    specs = [
        pl.BlockSpec((NODE_EMB, _TCB),
                     lambda i, k=k, m=max_blk: (0, jnp.minimum(4 * i + k, m)))
        for k in range(4)
    ]
    packed = pl.pallas_call(
        _transp_body,
        grid=(grid_n,),
        in_specs=specs,
        out_specs=pl.BlockSpec((_TCB, 4 * WPR), lambda i: (i, 0)),
        out_shape=jax.ShapeDtypeStruct((grid_n * _TCB, 4 * WPR), jnp.int32),
    )(tT, tT, tT, tT)
    return packed.reshape(grid_n * 4 * _TCB, WPR)


def _remap_idx(idx):
    r = idx & (4 * _TCB - 1)
    return (idx - r) + ((r & (_TCB - 1)) << 2) + (r >> _TCB_SHIFT)


def _sc_gather(n_idx_rows, n_tokens):
    info = plsc.get_sparse_core_info()
    nc, ns = info.num_cores, info.num_subcores
    nw = nc * ns
    rows_per_w = n_idx_rows // nw
    grps = rows_per_w // ROWS_PER_GRP
    half_w = nw // 2
    half_rows = n_idx_rows // 2
    mesh = plsc.VectorSubcoreMesh(core_axis_name="c", subcore_axis_name="s")

    @functools.partial(
        pl.kernel, mesh=mesh,
        out_type=jax.ShapeDtypeStruct((n_tokens // 2, 4 * WPR), jnp.int32),
        scratch_types=[
            pltpu.VMEM((ROWS_PER_GRP, LANES), jnp.int32),
            pltpu.VMEM((ROWS_PER_GRP, LANES), jnp.int32),
            pltpu.VMEM((GRP, WPR), jnp.int32),
            pltpu.VMEM((GRP, WPR), jnp.int32),
            pltpu.SemaphoreType.DMA,
        ],
        compiler_params=pltpu.CompilerParams(use_tc_tiling_on_sc=False),
    )
    def gather_k(vidx_hbm, eidx_hbm, ntab_hbm, etab_hbm,
                 cat_hbm, vidx, eidx, vrows, erows, sem):
        wid = lax.axis_index("s") * nc + lax.axis_index("c")
        row0 = wid * rows_per_w
        in_b = (wid >= half_w).astype(jnp.int32)
        col0 = in_b * (2 * WPR)
        out_row0 = (row0 - in_b * half_rows) * LANES

        def body(g, carry):
            r = row0 + g * ROWS_PER_GRP
            pltpu.sync_copy(vidx_hbm.at[pl.ds(r, ROWS_PER_GRP)], vidx)
            pltpu.sync_copy(eidx_hbm.at[pl.ds(r, ROWS_PER_GRP)], eidx)
            cps = []
            for j in range(ROWS_PER_GRP):
                cps.append(pltpu.async_copy(
                    ntab_hbm.at[vidx.at[j]],
                    vrows.at[pl.ds(j * LANES, LANES)], sem))
                cps.append(pltpu.async_copy(
                    etab_hbm.at[eidx.at[j]],
                    erows.at[pl.ds(j * LANES, LANES)], sem))
            for c in cps:
                c.wait()
            tok = out_row0 + g * GRP
            pltpu.sync_copy(
                vrows, cat_hbm.at[pl.ds(tok, GRP), pl.ds(col0, WPR)])
            pltpu.sync_copy(
                erows, cat_hbm.at[pl.ds(tok, GRP), pl.ds(col0 + WPR, WPR)])
            return carry

        lax.fori_loop(0, grps, body, 0)

    return gather_k


def _unpack_dot(x, wlo_ref, whi_ref, b_ref):
    lo = lax.bitcast_convert_type(x << 16, jnp.float32)
    hi = lax.bitcast_convert_type(x & jnp.int32(-65536), jnp.float32)
    h = jnp.dot(lo, wlo_ref[...], preferred_element_type=jnp.float32)
    h = h + jnp.dot(hi, whi_ref[...], preferred_element_type=jnp.float32)
    return h + b_ref[...]


def _tc_compute(x, wlo_ref, whi_ref, b_ref, pe_ref, h_ref, hp_ref):
    h = _unpack_dot(x, wlo_ref, whi_ref, b_ref)
    h_ref[...] = h
    hp_ref[...] = h + pe_ref[...]


def _tc_body_a0(cat_ref, wlo_ref, whi_ref, b_ref, pe_ref, h_ref, hp_ref):
    _tc_compute(cat_ref[:, 0:2 * WPR], wlo_ref, whi_ref, b_ref, pe_ref,
                h_ref, hp_ref)


def _tc_body_a(cat_ref, wlo_ref, whi_ref, b_ref, pe_ref, hin, hpin,
               h_ref, hp_ref):
    del hin, hpin
    _tc_compute(cat_ref[:, 0:2 * WPR], wlo_ref, whi_ref, b_ref, pe_ref,
                h_ref, hp_ref)


def _tc_body_b(cat_ref, wlo_ref, whi_ref, b_ref, pe_ref, hin, hpin,
               h_ref, hp_ref):
    del hin, hpin
    _tc_compute(cat_ref[:, 2 * WPR:], wlo_ref, whi_ref, b_ref, pe_ref,
                h_ref, hp_ref)


def kernel(v_list, e_list, node_table, edge_table, W, b):
    B, L = v_list.shape
    n_tokens = B * L
    n_idx_rows = n_tokens // LANES

    nt_lin = _relayout_table(node_table)
    et_lin = _relayout_table(edge_table)

    v2d = _remap_idx(v_list.reshape(n_idx_rows, LANES))
    e2d = _remap_idx(e_list.reshape(n_idx_rows, LANES))

    rows_c = n_idx_rows // N_CHUNKS
    tok_c = n_tokens // N_CHUNKS
    nblk_half = tok_c // 2 // RB

    gk = _sc_gather(rows_c, tok_c)
    cats = [
        gk(v2d[k * rows_c:(k + 1) * rows_c],
           e2d[k * rows_c:(k + 1) * rows_c],
           nt_lin, et_lin)
        for k in range(N_CHUNKS)
    ]

    w2 = (np.sqrt(float(NODE_EMB)) * W).T
    lo_rows = np.r_[0:WPR, NODE_EMB:NODE_EMB + WPR]
    hi_rows = np.r_[WPR:NODE_EMB, NODE_EMB + WPR:2 * NODE_EMB]
    wlo = w2[lo_rows, :]
    whi = w2[hi_rows, :]
    b2 = b.reshape(1, D_MODEL)
    pe_tile = jnp.asarray(np.tile(_pe_table(L, D_MODEL), (BB, 1)))

    out_shape = [
        jax.ShapeDtypeStruct((n_tokens, D_MODEL), jnp.float32),
        jax.ShapeDtypeStruct((n_tokens, D_MODEL), jnp.float32),
    ]
    common_specs = [
        pl.BlockSpec((NODE_EMB, D_MODEL), lambda i: (0, 0)),
        pl.BlockSpec((NODE_EMB, D_MODEL), lambda i: (0, 0)),
        pl.BlockSpec((1, D_MODEL), lambda i: (0, 0)),
        pl.BlockSpec((RB, D_MODEL), lambda i: (0, 0)),
    ]
    alias_specs = [pl.BlockSpec(memory_space=pl.ANY),
                   pl.BlockSpec(memory_space=pl.ANY)]

    h_emb = h_pos = None
    for k in range(N_CHUNKS):
        for phase, body in ((0, _tc_body_a if k else _tc_body_a0),
                            (1, _tc_body_b)):
            off = (2 * k + phase) * nblk_half
            first = h_emb is None
            outs = pl.pallas_call(
                body,
                grid=(nblk_half,),
                in_specs=[pl.BlockSpec((RB, D_MODEL), lambda i: (i, 0))]
                + common_specs + ([] if first else alias_specs),
                out_specs=[
                    pl.BlockSpec((RB, D_MODEL),
                                 lambda i, off=off: (i + off, 0)),
                    pl.BlockSpec((RB, D_MODEL),
                                 lambda i, off=off: (i + off, 0)),
                ],
                out_shape=out_shape,
                input_output_aliases={} if first else {5: 0, 6: 1},
            )(*([cats[k], wlo, whi, b2, pe_tile]
                + ([] if first else [h_emb, h_pos])))
            h_emb, h_pos = outs

    return (h_emb.reshape(B, L, D_MODEL), h_pos.reshape(B, L, D_MODEL))


# speedup vs baseline: 2.1070x; 1.0349x over previous
import functools

import numpy as np
import jax
import jax.numpy as jnp
from jax import lax
from jax.experimental import pallas as pl
from jax.experimental.pallas import tpu as pltpu
from jax.experimental.pallas import tpu_sc as plsc

NODE_EMB = 64
EDGE_EMB = 64
D_MODEL = 128
WPR = NODE_EMB // 2
LANES = 128
ROWS_PER_GRP = 5
GRP = LANES * ROWS_PER_GRP
N_CHUNKS = 4
BB = 32
RB = BB * 200
_TCB = 4096
_TCB_SHIFT = _TCB.bit_length() - 1


def _pe_table(seq_len, d_model):
    pos = np.arange(seq_len, dtype=np.float32)[:, None]
    div = np.exp(np.arange(0, d_model, 2, dtype=np.float32)
                 * (-np.log(10000.0) / d_model))
    pe = np.zeros((seq_len, d_model), dtype=np.float32)
    pe[:, 0::2] = np.sin(pos * div)
    pe[:, 1::2] = np.cos(pos * div)
    return pe


def _rne_bf16(x):
    u = lax.bitcast_convert_type(x, jnp.int32)
    return lax.shift_right_logical(
        u + 0x7FFF + (lax.shift_right_logical(u, 16) & 1), 16)


def _transp_body(a0, a1, a2, a3, o_ref):
    for k, a in enumerate((a0, a1, a2, a3)):
        blk = a[...]
        word = _rne_bf16(blk[:WPR, :]) | (_rne_bf16(blk[WPR:, :]) << 16)
        o_ref[:, k * WPR:(k + 1) * WPR] = word.T


def _relayout_table(table):
    V = table.shape[0]
    grid_n = (V + 4 * _TCB - 1) // (4 * _TCB)
    max_blk = (V + _TCB - 1) // _TCB - 1
    tT = table.T
    specs = [
        pl.BlockSpec((NODE_EMB, _TCB),
                     lambda i, k=k, m=max_blk: (0, jnp.minimum(4 * i + k, m)))
        for k in range(4)
    ]
    packed = pl.pallas_call(
        _transp_body,
        grid=(grid_n,),
        in_specs=specs,
        out_specs=pl.BlockSpec((_TCB, 4 * WPR), lambda i: (i, 0)),
        out_shape=jax.ShapeDtypeStruct((grid_n * _TCB, 4 * WPR), jnp.int32),
    )(tT, tT, tT, tT)
    return packed.reshape(grid_n * 4 * _TCB, WPR)


def _remap_idx(idx):
    r = idx & (4 * _TCB - 1)
    return (idx - r) + ((r & (_TCB - 1)) << 2) + (r >> _TCB_SHIFT)


def _sc_gather(n_idx_rows, n_tokens):
    info = plsc.get_sparse_core_info()
    nc, ns = info.num_cores, info.num_subcores
    nw = nc * ns
    rows_per_w = n_idx_rows // nw
    grps = rows_per_w // ROWS_PER_GRP
    half_w = nw // 2
    half_rows = n_idx_rows // 2
    mesh = plsc.VectorSubcoreMesh(core_axis_name="c", subcore_axis_name="s")

    @functools.partial(
        pl.kernel, mesh=mesh,
        out_type=jax.ShapeDtypeStruct((n_tokens // 2, 4 * WPR), jnp.int32),
        scratch_types=[
            pltpu.VMEM((ROWS_PER_GRP, LANES), jnp.int32),
            pltpu.VMEM((ROWS_PER_GRP, LANES), jnp.int32),
            pltpu.VMEM((GRP, WPR), jnp.int32),
            pltpu.VMEM((GRP, WPR), jnp.int32),
            pltpu.SemaphoreType.DMA,
        ],
        compiler_params=pltpu.CompilerParams(use_tc_tiling_on_sc=False),
    )
    def gather_k(vidx_hbm, eidx_hbm, ntab_hbm, etab_hbm,
                 cat_hbm, vidx, eidx, vrows, erows, sem):
        wid = lax.axis_index("s") * nc + lax.axis_index("c")
        row0 = wid * rows_per_w
        in_b = (wid >= half_w).astype(jnp.int32)
        col0 = in_b * (2 * WPR)
        out_row0 = (row0 - in_b * half_rows) * LANES

        def body(g, carry):
            r = row0 + g * ROWS_PER_GRP
            pltpu.sync_copy(vidx_hbm.at[pl.ds(r, ROWS_PER_GRP)], vidx)
            pltpu.sync_copy(eidx_hbm.at[pl.ds(r, ROWS_PER_GRP)], eidx)
            cps = []
            for j in range(ROWS_PER_GRP):
                cps.append(pltpu.async_copy(
                    ntab_hbm.at[vidx.at[j]],
                    vrows.at[pl.ds(j * LANES, LANES)], sem))
                cps.append(pltpu.async_copy(
                    etab_hbm.at[eidx.at[j]],
                    erows.at[pl.ds(j * LANES, LANES)], sem))
            for c in cps:
                c.wait()
            tok = out_row0 + g * GRP
            pltpu.sync_copy(
                vrows, cat_hbm.at[pl.ds(tok, GRP), pl.ds(col0, WPR)])
            pltpu.sync_copy(
                erows, cat_hbm.at[pl.ds(tok, GRP), pl.ds(col0 + WPR, WPR)])
            return carry

        lax.fori_loop(0, grps, body, 0)

    return gather_k


def _unpack_dot(x, wlo_ref, whi_ref, b_ref):
    lo = lax.bitcast_convert_type(x << 16, jnp.float32)
    hi = lax.bitcast_convert_type(x & jnp.int32(-65536), jnp.float32)
    h = jnp.dot(lo, wlo_ref[...], preferred_element_type=jnp.float32)
    h = h + jnp.dot(hi, whi_ref[...], preferred_element_type=jnp.float32)
    return h + b_ref[...]


def _tc_compute(x, wlo_ref, whi_ref, b_ref, pe_ref, h_ref, hp_ref):
    h = _unpack_dot(x, wlo_ref, whi_ref, b_ref)
    h_ref[...] = h
    hp_ref[...] = h + pe_ref[...]


def _tc_body_a0(cat_ref, wlo_ref, whi_ref, b_ref, pe_ref, h_ref, hp_ref):
    _tc_compute(cat_ref[:, 0:2 * WPR], wlo_ref, whi_ref, b_ref, pe_ref,
                h_ref, hp_ref)


def _tc_body_a(cat_ref, wlo_ref, whi_ref, b_ref, pe_ref, hin, hpin,
               h_ref, hp_ref):
    del hin, hpin
    _tc_compute(cat_ref[:, 0:2 * WPR], wlo_ref, whi_ref, b_ref, pe_ref,
                h_ref, hp_ref)


def _tc_body_b(cat_ref, wlo_ref, whi_ref, b_ref, pe_ref, hin, hpin,
               h_ref, hp_ref):
    del hin, hpin
    _tc_compute(cat_ref[:, 2 * WPR:], wlo_ref, whi_ref, b_ref, pe_ref,
                h_ref, hp_ref)


def kernel(v_list, e_list, node_table, edge_table, W, b):
    B, L = v_list.shape
    n_tokens = B * L
    n_idx_rows = n_tokens // LANES

    nt_lin = _relayout_table(node_table)
    et_lin = _relayout_table(edge_table)

    v2d = _remap_idx(v_list.reshape(n_idx_rows, LANES))
    e2d = _remap_idx(e_list.reshape(n_idx_rows, LANES))

    rows_c = n_idx_rows // N_CHUNKS
    tok_c = n_tokens // N_CHUNKS
    nblk_half = tok_c // 2 // RB

    gk = _sc_gather(rows_c, tok_c)
    cats = [
        gk(v2d[k * rows_c:(k + 1) * rows_c],
           e2d[k * rows_c:(k + 1) * rows_c],
           nt_lin, et_lin)
        for k in range(N_CHUNKS)
    ]

    w2 = (np.sqrt(float(NODE_EMB)) * W).T
    lo_rows = np.r_[0:WPR, NODE_EMB:NODE_EMB + WPR]
    hi_rows = np.r_[WPR:NODE_EMB, NODE_EMB + WPR:2 * NODE_EMB]
    wlo = w2[lo_rows, :]
    whi = w2[hi_rows, :]
    b2 = b.reshape(1, D_MODEL)
    pe_tile = jnp.asarray(np.tile(_pe_table(L, D_MODEL), (BB, 1)))

    out_shape = [
        jax.ShapeDtypeStruct((n_tokens, D_MODEL), jnp.float32),
        jax.ShapeDtypeStruct((n_tokens, D_MODEL), jnp.float32),
    ]
    common_specs = [
        pl.BlockSpec((NODE_EMB, D_MODEL), lambda i: (0, 0)),
        pl.BlockSpec((NODE_EMB, D_MODEL), lambda i: (0, 0)),
        pl.BlockSpec((1, D_MODEL), lambda i: (0, 0)),
        pl.BlockSpec((RB, D_MODEL), lambda i: (0, 0)),
    ]
    alias_specs = [pl.BlockSpec(memory_space=pl.ANY),
                   pl.BlockSpec(memory_space=pl.ANY)]

    h_emb = h_pos = None
    for k in range(N_CHUNKS):
        for phase, body in ((0, _tc_body_a if k else _tc_body_a0),
                            (1, _tc_body_b)):
            off = (2 * k + phase) * nblk_half
            first = h_emb is None
            outs = pl.pallas_call(
                body,
                grid=(nblk_half,),
                in_specs=[pl.BlockSpec((RB, D_MODEL), lambda i: (i, 0))]
                + common_specs + ([] if first else alias_specs),
                out_specs=[
                    pl.BlockSpec((RB, D_MODEL),
                                 lambda i, off=off: (i + off, 0)),
                    pl.BlockSpec((RB, D_MODEL),
                                 lambda i, off=off: (i + off, 0)),
                ],
                out_shape=out_shape,
                input_output_aliases={} if first else {5: 0, 6: 1},
            )(*([cats[k], wlo, whi, b2, pe_tile]
                + ([] if first else [h_emb, h_pos])))
            h_emb, h_pos = outs

    return (h_emb.reshape(B, L, D_MODEL), h_pos.reshape(B, L, D_MODEL))


# speedup vs baseline: 2.3133x; 1.0979x over previous
import functools

import numpy as np
import jax
import jax.numpy as jnp
from jax import lax
from jax.experimental import pallas as pl
from jax.experimental.pallas import tpu as pltpu
from jax.experimental.pallas import tpu_sc as plsc

NODE_EMB = 64
EDGE_EMB = 64
D_MODEL = 128
WPR = NODE_EMB // 2
LANES = 128
ROWS_PER_GRP = 5
GRP = LANES * ROWS_PER_GRP
N_CHUNKS = 4
BB = 32
RB = BB * 200
_TCB = 4096
_TCB_SHIFT = _TCB.bit_length() - 1


def _pe_table(seq_len, d_model):
    pos = np.arange(seq_len, dtype=np.float32)[:, None]
    div = np.exp(np.arange(0, d_model, 2, dtype=np.float32)
                 * (-np.log(10000.0) / d_model))
    pe = np.zeros((seq_len, d_model), dtype=np.float32)
    pe[:, 0::2] = np.sin(pos * div)
    pe[:, 1::2] = np.cos(pos * div)
    return pe


def _rne_bf16(x):
    u = lax.bitcast_convert_type(x, jnp.int32)
    return lax.shift_right_logical(
        u + 0x7FFF + (lax.shift_right_logical(u, 16) & 1), 16)


def _transp_body(a0, a1, a2, a3, o_ref):
    for k, a in enumerate((a0, a1, a2, a3)):
        blk = a[...]
        word = _rne_bf16(blk[:WPR, :]) | (_rne_bf16(blk[WPR:, :]) << 16)
        o_ref[:, k * WPR:(k + 1) * WPR] = word.T


def _relayout_table(table):
    V = table.shape[0]
    grid_n = (V + 4 * _TCB - 1) // (4 * _TCB)
    max_blk = (V + _TCB - 1) // _TCB - 1
    tT = table.T
    specs = [
        pl.BlockSpec((NODE_EMB, _TCB),
                     lambda i, k=k, m=max_blk: (0, jnp.minimum(4 * i + k, m)))
        for k in range(4)
    ]
    packed = pl.pallas_call(
        _transp_body,
        grid=(grid_n,),
        in_specs=specs,
        out_specs=pl.BlockSpec((_TCB, 4 * WPR), lambda i: (i, 0)),
        out_shape=jax.ShapeDtypeStruct((grid_n * _TCB, 4 * WPR), jnp.int32),
    )(tT, tT, tT, tT)
    return packed.reshape(grid_n * 4 * _TCB, WPR)


def _remap_idx(idx):
    r = idx & (4 * _TCB - 1)
    return (idx - r) + ((r & (_TCB - 1)) << 2) + (r >> _TCB_SHIFT)


def _sc_gather(n_idx_rows, n_tokens):
    info = plsc.get_sparse_core_info()
    nc, ns = info.num_cores, info.num_subcores
    nw = nc * ns
    rows_per_w = n_idx_rows // nw
    grps = rows_per_w // ROWS_PER_GRP
    grps_half = grps // 2
    cat_per_w = rows_per_w * LANES // 2
    mesh = plsc.VectorSubcoreMesh(core_axis_name="c", subcore_axis_name="s")

    @functools.partial(
        pl.kernel, mesh=mesh,
        out_type=jax.ShapeDtypeStruct((n_tokens // 2, 4 * WPR), jnp.int32),
        scratch_types=[
            pltpu.VMEM((ROWS_PER_GRP, LANES), jnp.int32),
            pltpu.VMEM((ROWS_PER_GRP, LANES), jnp.int32),
            pltpu.VMEM((GRP, WPR), jnp.int32),
            pltpu.VMEM((GRP, WPR), jnp.int32),
            pltpu.SemaphoreType.DMA,
        ],
        compiler_params=pltpu.CompilerParams(use_tc_tiling_on_sc=False),
    )
    def gather_k(vidx_hbm, eidx_hbm, ntab_hbm, etab_hbm,
                 cat_hbm, vidx, eidx, vrows, erows, sem):
        wid = lax.axis_index("s") * nc + lax.axis_index("c")
        row0 = wid * rows_per_w
        cat0 = wid * cat_per_w

        def body(g, carry):
            r = row0 + g * ROWS_PER_GRP
            pltpu.sync_copy(vidx_hbm.at[pl.ds(r, ROWS_PER_GRP)], vidx)
            pltpu.sync_copy(eidx_hbm.at[pl.ds(r, ROWS_PER_GRP)], eidx)
            cps = []
            for j in range(ROWS_PER_GRP):
                cps.append(pltpu.async_copy(
                    ntab_hbm.at[vidx.at[j]],
                    vrows.at[pl.ds(j * LANES, LANES)], sem))
                cps.append(pltpu.async_copy(
                    etab_hbm.at[eidx.at[j]],
                    erows.at[pl.ds(j * LANES, LANES)], sem))
            for c in cps:
                c.wait()
            col0 = (g // grps_half) * (2 * WPR)
            tok = cat0 + (g % grps_half) * GRP
            pltpu.sync_copy(
                vrows, cat_hbm.at[pl.ds(tok, GRP), pl.ds(col0, WPR)])
            pltpu.sync_copy(
                erows, cat_hbm.at[pl.ds(tok, GRP), pl.ds(col0 + WPR, WPR)])
            return carry

        lax.fori_loop(0, grps, body, 0)

    return gather_k


def _unpack_dot(x, wlo_ref, whi_ref, b_ref):
    lo = lax.bitcast_convert_type(x << 16, jnp.float32)
    hi = lax.bitcast_convert_type(x & jnp.int32(-65536), jnp.float32)
    h = jnp.dot(lo, wlo_ref[...], preferred_element_type=jnp.float32)
    h = h + jnp.dot(hi, whi_ref[...], preferred_element_type=jnp.float32)
    return h + b_ref[...]


def _tc_compute(cat_ref, wlo_ref, whi_ref, b_ref, pe_ref, h_ref, hp_ref):
    x = cat_ref[...]
    rbh = x.shape[0]
    ha = _unpack_dot(x[:, 0:2 * WPR], wlo_ref, whi_ref, b_ref)
    hb = _unpack_dot(x[:, 2 * WPR:], wlo_ref, whi_ref, b_ref)
    pe = pe_ref[...]
    h_ref[0:rbh] = ha
    h_ref[rbh:] = hb
    hp_ref[0:rbh] = ha + pe
    hp_ref[rbh:] = hb + pe


def _tc_body_first(cat_ref, wlo_ref, whi_ref, b_ref, pe_ref, h_ref, hp_ref):
    _tc_compute(cat_ref, wlo_ref, whi_ref, b_ref, pe_ref, h_ref, hp_ref)


def _tc_body_next(cat_ref, wlo_ref, whi_ref, b_ref, pe_ref, hin, hpin,
                  h_ref, hp_ref):
    del hin, hpin
    _tc_compute(cat_ref, wlo_ref, whi_ref, b_ref, pe_ref, h_ref, hp_ref)


def kernel(v_list, e_list, node_table, edge_table, W, b):
    B, L = v_list.shape
    n_tokens = B * L
    n_idx_rows = n_tokens // LANES

    nt_lin = _relayout_table(node_table)
    et_lin = _relayout_table(edge_table)

    v2d = _remap_idx(v_list.reshape(n_idx_rows, LANES))
    e2d = _remap_idx(e_list.reshape(n_idx_rows, LANES))

    rows_c = n_idx_rows // N_CHUNKS
    tok_c = n_tokens // N_CHUNKS
    RBH = RB // 2
    nblk = tok_c // RB

    gk = _sc_gather(rows_c, tok_c)
    cats = [
        gk(v2d[k * rows_c:(k + 1) * rows_c],
           e2d[k * rows_c:(k + 1) * rows_c],
           nt_lin, et_lin)
        for k in range(N_CHUNKS)
    ]

    w2 = (np.sqrt(float(NODE_EMB)) * W).T
    lo_rows = np.r_[0:WPR, NODE_EMB:NODE_EMB + WPR]
    hi_rows = np.r_[WPR:NODE_EMB, NODE_EMB + WPR:2 * NODE_EMB]
    wlo = w2[lo_rows, :]
    whi = w2[hi_rows, :]
    b2 = b.reshape(1, D_MODEL)
    pe_tile = jnp.asarray(
        np.tile(_pe_table(L, D_MODEL), (RBH // L, 1)))

    out_shape = [
        jax.ShapeDtypeStruct((n_tokens, D_MODEL), jnp.float32),
        jax.ShapeDtypeStruct((n_tokens, D_MODEL), jnp.float32),
    ]
    common_specs = [
        pl.BlockSpec((NODE_EMB, D_MODEL), lambda i: (0, 0)),
        pl.BlockSpec((NODE_EMB, D_MODEL), lambda i: (0, 0)),
        pl.BlockSpec((1, D_MODEL), lambda i: (0, 0)),
        pl.BlockSpec((RBH, D_MODEL), lambda i: (0, 0)),
    ]
    alias_specs = [pl.BlockSpec(memory_space=pl.ANY),
                   pl.BlockSpec(memory_space=pl.ANY)]

    h_emb = h_pos = None
    for k in range(N_CHUNKS):
        off = k * nblk
        first = h_emb is None
        h_emb, h_pos = pl.pallas_call(
            _tc_body_first if first else _tc_body_next,
            grid=(nblk,),
            in_specs=[pl.BlockSpec((RBH, D_MODEL), lambda i: (i, 0))]
            + common_specs + ([] if first else alias_specs),
            out_specs=[
                pl.BlockSpec((RB, D_MODEL), lambda i, off=off: (i + off, 0)),
                pl.BlockSpec((RB, D_MODEL), lambda i, off=off: (i + off, 0)),
            ],
            out_shape=out_shape,
            input_output_aliases={} if first else {5: 0, 6: 1},
        )(*([cats[k], wlo, whi, b2, pe_tile]
            + ([] if first else [h_emb, h_pos])))

    return (h_emb.reshape(B, L, D_MODEL), h_pos.reshape(B, L, D_MODEL))
